# Initial kernel scaffold; baseline (speedup 1.0000x reference)
import jax
import jax.numpy as jnp
from jax.experimental import pallas as pl


def kernel(x):
    raise NotImplementedError("write your pallas kernel here")


# speedup vs baseline: 3.1341x; 3.1341x over previous
import functools

import jax
import jax.numpy as jnp
from jax.experimental import pallas as pl

K = 16
KK = 32


def _knn_body(x_ref, out_ref):
    xb = x_ref[0]
    C, N = xb.shape
    norm = jnp.sqrt(jnp.sum(xb * xb, axis=0, keepdims=True))
    xn = xb / jnp.maximum(norm, 1e-12)
    inner = jax.lax.dot_general(
        xn, xn,
        dimension_numbers=(((0,), (0,)), ((), ())),
        preferred_element_type=jnp.float32,
    )
    x_inner = -2.0 * inner
    sq = jnp.sum(xn * xn, axis=0, keepdims=True)
    dist = (jnp.transpose(sq) + x_inner) + sq
    score = -dist
    col = jax.lax.broadcasted_iota(jnp.int32, (N, N), 1)
    neg_inf = jnp.float32(-jnp.inf)
    cols_out = []
    for k in range(KK):
        m = jnp.max(score, axis=1, keepdims=True)
        eq = score == m
        idx = jnp.min(jnp.where(eq, col, N), axis=1, keepdims=True)
        if k % 2 == 0:
            cols_out.append(idx)
        if k != KK - 1:
            score = jnp.where(col == idx, neg_inf, score)
    out_ref[0] = jnp.concatenate(cols_out, axis=1)


@jax.jit
def kernel(x):
    B, C, N, _ = x.shape
    xs = jnp.squeeze(x, -1)
    nn_idx = pl.pallas_call(
        _knn_body,
        grid=(B,),
        in_specs=[pl.BlockSpec((1, C, N), lambda b: (b, 0, 0))],
        out_specs=pl.BlockSpec((1, N, K), lambda b: (b, 0, 0)),
        out_shape=jax.ShapeDtypeStruct((B, N, K), jnp.int32),
    )(xs)
    center_idx = jnp.broadcast_to(
        jnp.arange(N, dtype=jnp.int32)[None, :, None], (B, N, K)
    )
    return jnp.stack((nn_idx, center_idx), axis=0)
